# baseline (speedup 1.0000x reference)
import jax
import jax.numpy as jnp
from jax.experimental import pallas as pl
from jax.experimental.pallas import tpu as pltpu

D_MODEL = 2048
NUM_EXPERTS = 64
MAX_SEQ_LEN = 4096
SPREAD = 2.0

TC = 128
LANES = 128


def _router_kernel(x_ref, sig_ref, idx_ref, soft_ref):
    i = pl.program_id(0)
    B = x_ref.shape[0]
    R = B * TC
    x = x_ref[...].reshape(R, D_MODEL)
    sigs = jnp.sign(sig_ref[...])
    scores = jax.lax.dot_general(
        x, sigs, (((1,), (1,)), ((), ())),
        preferred_element_type=jnp.float32)

    scores_t = jnp.transpose(scores)

    lane = jax.lax.broadcasted_iota(jnp.int32, (NUM_EXPERTS, R), 1)
    t = (i * TC + (lane & (TC - 1))).astype(jnp.float32)
    centers = jax.lax.broadcasted_iota(
        jnp.int32, (NUM_EXPERTS, R), 0).astype(jnp.float32)
    u = (t * (NUM_EXPERTS / MAX_SEQ_LEN) - centers) * (1.0 / SPREAD)
    a = jnp.abs(u)
    pos = jnp.where(
        a < 1.0, 2.0 / 3.0 - a * a + 0.5 * a * a * a,
        jnp.where(a < 2.0, (1.0 / 6.0) * (2.0 - a) ** 3, 0.0))

    combined = scores_t * pos

    m = jnp.max(combined, axis=0, keepdims=True)
    e = jnp.exp(combined - m)
    s = jnp.sum(e, axis=0, keepdims=True)
    sm = e / s

    row = jax.lax.broadcasted_iota(
        jnp.int32, (NUM_EXPERTS, R), 0).astype(jnp.float32)
    cand = jnp.where(combined == m, row, float(NUM_EXPERTS))
    c = jnp.min(cand, axis=0, keepdims=True)

    for b in range(B):
        soft_ref[b, :, :] = sm[:, b * TC:(b + 1) * TC]
        idx_ref[b:b + 1, :] = c[:, b * TC:(b + 1) * TC].astype(jnp.int32)


def kernel(x, signatures):
    B, T, D = x.shape
    grid = (T // TC,)
    idx, soft = pl.pallas_call(
        _router_kernel,
        grid=grid,
        in_specs=[
            pl.BlockSpec((B, TC, D), lambda i: (0, i, 0)),
            pl.BlockSpec((NUM_EXPERTS, D), lambda i: (0, 0)),
        ],
        out_specs=[
            pl.BlockSpec((B, TC), lambda i: (0, i)),
            pl.BlockSpec((B, NUM_EXPERTS, TC), lambda i: (0, 0, i)),
        ],
        out_shape=[
            jax.ShapeDtypeStruct((B, T), jnp.int32),
            jax.ShapeDtypeStruct((B, NUM_EXPERTS, T), jnp.float32),
        ],
        compiler_params=pltpu.CompilerParams(
            dimension_semantics=("parallel",),
        ),
    )(x, signatures)
    return idx, jnp.swapaxes(soft, 1, 2)


# speedup vs baseline: 1.1862x; 1.1862x over previous
import jax
import jax.numpy as jnp
from jax.experimental import pallas as pl
from jax.experimental.pallas import tpu as pltpu

D_MODEL = 2048
NUM_EXPERTS = 64
MAX_SEQ_LEN = 4096
SPREAD = 2.0

TC = 256
LANES = 128


def _router_kernel(x_ref, sig_ref, idx_ref, soft_ref):
    i = pl.program_id(0)
    B = x_ref.shape[0]
    R = B * TC
    x = x_ref[...].reshape(R, D_MODEL)
    sigs = jnp.sign(sig_ref[...])
    scores = jax.lax.dot_general(
        x, sigs, (((1,), (1,)), ((), ())),
        preferred_element_type=jnp.float32)

    scores_t = jnp.transpose(scores)

    lane = jax.lax.broadcasted_iota(jnp.int32, (NUM_EXPERTS, R), 1)
    t = (i * TC + (lane & (TC - 1))).astype(jnp.float32)
    centers = jax.lax.broadcasted_iota(
        jnp.int32, (NUM_EXPERTS, R), 0).astype(jnp.float32)
    u = (t * (NUM_EXPERTS / MAX_SEQ_LEN) - centers) * (1.0 / SPREAD)
    a = jnp.abs(u)
    pos = jnp.where(
        a < 1.0, 2.0 / 3.0 - a * a + 0.5 * a * a * a,
        jnp.where(a < 2.0, (1.0 / 6.0) * (2.0 - a) ** 3, 0.0))

    combined = scores_t * pos

    m = jnp.max(combined, axis=0, keepdims=True)
    e = jnp.exp(combined - m)
    s = jnp.sum(e, axis=0, keepdims=True)
    sm = e / s

    row = jax.lax.broadcasted_iota(
        jnp.int32, (NUM_EXPERTS, R), 0).astype(jnp.float32)
    cand = jnp.where(combined == m, row, float(NUM_EXPERTS))
    c = jnp.min(cand, axis=0, keepdims=True)

    for b in range(B):
        soft_ref[b, :, :] = sm[:, b * TC:(b + 1) * TC]
        idx_ref[b:b + 1, :] = c[:, b * TC:(b + 1) * TC].astype(jnp.int32)


def kernel(x, signatures):
    B, T, D = x.shape
    grid = (T // TC,)
    idx, soft = pl.pallas_call(
        _router_kernel,
        grid=grid,
        in_specs=[
            pl.BlockSpec((B, TC, D), lambda i: (0, i, 0)),
            pl.BlockSpec((NUM_EXPERTS, D), lambda i: (0, 0)),
        ],
        out_specs=[
            pl.BlockSpec((B, TC), lambda i: (0, i)),
            pl.BlockSpec((B, NUM_EXPERTS, TC), lambda i: (0, 0, i)),
        ],
        out_shape=[
            jax.ShapeDtypeStruct((B, T), jnp.int32),
            jax.ShapeDtypeStruct((B, NUM_EXPERTS, T), jnp.float32),
        ],
        compiler_params=pltpu.CompilerParams(
            dimension_semantics=("parallel",),
        ),
    )(x, signatures)
    return idx, jnp.swapaxes(soft, 1, 2)
